# baseline (speedup 1.0000x reference)
import functools

import jax
import jax.numpy as jnp
from jax.experimental import pallas as pl

_FILTERS = [64, 128, 256, 512]
_CHANNELS = [2, 2, 2, 2]
_N1, _E1, _NB = 2000, 8000, 8


def _mm1_body(x_ref, w_ref, b_ref, o_ref, *, relu):
    y = jnp.dot(x_ref[...], w_ref[...], preferred_element_type=jnp.float32)
    y = y + b_ref[...]
    if relu:
        y = jnp.maximum(y, 0.0)
    o_ref[...] = y


def _mm2_body(x1_ref, w1_ref, x2_ref, w2_ref, b_ref, o_ref, *, relu):
    y = jnp.dot(x1_ref[...], w1_ref[...], preferred_element_type=jnp.float32)
    y = y + jnp.dot(x2_ref[...], w2_ref[...], preferred_element_type=jnp.float32)
    y = y + b_ref[...]
    if relu:
        y = jnp.maximum(y, 0.0)
    o_ref[...] = y


def _mm_add_body(x_ref, w_ref, add_ref, b_ref, o_ref, *, relu):
    y = jnp.dot(x_ref[...], w_ref[...], preferred_element_type=jnp.float32)
    y = y + add_ref[...] + b_ref[...]
    if relu:
        y = jnp.maximum(y, 0.0)
    o_ref[...] = y


def _affine_relu_body(y_ref, a_ref, c_ref, o_ref):
    o_ref[...] = jnp.maximum(y_ref[...] * a_ref[...] + c_ref[...], 0.0)


def _att_body(x_ref, z_ref, a1_ref, o_ref):
    logit = jnp.dot(x_ref[...], a1_ref[...], preferred_element_type=jnp.float32)
    logit = logit + z_ref[...]
    att = 0.5 + 0.5 * jax.nn.sigmoid(logit)
    o_ref[...] = x_ref[...] * att


def _rows_block(n):
    return 1000 if n % 1000 == 0 else n


def _pmm(x, w, b, relu=False):
    n, k = x.shape
    m = w.shape[1]
    br = _rows_block(n)
    return pl.pallas_call(
        functools.partial(_mm1_body, relu=relu),
        grid=(n // br,),
        in_specs=[
            pl.BlockSpec((br, k), lambda i: (i, 0)),
            pl.BlockSpec((k, m), lambda i: (0, 0)),
            pl.BlockSpec((1, m), lambda i: (0, 0)),
        ],
        out_specs=pl.BlockSpec((br, m), lambda i: (i, 0)),
        out_shape=jax.ShapeDtypeStruct((n, m), jnp.float32),
    )(x, w, b.reshape(1, -1))


def _pmm2(x1, w1, x2, w2, b, relu=False):
    n, k1 = x1.shape
    k2 = x2.shape[1]
    m = w1.shape[1]
    br = _rows_block(n)
    return pl.pallas_call(
        functools.partial(_mm2_body, relu=relu),
        grid=(n // br,),
        in_specs=[
            pl.BlockSpec((br, k1), lambda i: (i, 0)),
            pl.BlockSpec((k1, m), lambda i: (0, 0)),
            pl.BlockSpec((br, k2), lambda i: (i, 0)),
            pl.BlockSpec((k2, m), lambda i: (0, 0)),
            pl.BlockSpec((1, m), lambda i: (0, 0)),
        ],
        out_specs=pl.BlockSpec((br, m), lambda i: (i, 0)),
        out_shape=jax.ShapeDtypeStruct((n, m), jnp.float32),
    )(x1, w1, x2, w2, b.reshape(1, -1))


def _pmm_add(x, w, add, b, relu=False):
    n, k = x.shape
    m = w.shape[1]
    br = _rows_block(n)
    return pl.pallas_call(
        functools.partial(_mm_add_body, relu=relu),
        grid=(n // br,),
        in_specs=[
            pl.BlockSpec((br, k), lambda i: (i, 0)),
            pl.BlockSpec((k, m), lambda i: (0, 0)),
            pl.BlockSpec((br, m), lambda i: (i, 0)),
            pl.BlockSpec((1, m), lambda i: (0, 0)),
        ],
        out_specs=pl.BlockSpec((br, m), lambda i: (i, 0)),
        out_shape=jax.ShapeDtypeStruct((n, m), jnp.float32),
    )(x, w, add, b.reshape(1, -1))


def _paffine_relu(y, a, c):
    n, m = y.shape
    br = _rows_block(n)
    return pl.pallas_call(
        _affine_relu_body,
        grid=(n // br,),
        in_specs=[
            pl.BlockSpec((br, m), lambda i: (i, 0)),
            pl.BlockSpec((1, m), lambda i: (0, 0)),
            pl.BlockSpec((1, m), lambda i: (0, 0)),
        ],
        out_specs=pl.BlockSpec((br, m), lambda i: (i, 0)),
        out_shape=jax.ShapeDtypeStruct((n, m), jnp.float32),
    )(y, a.reshape(1, -1), c.reshape(1, -1))


def _patt_scale(x, z, a1):
    n, d = x.shape
    br = _rows_block(n)
    return pl.pallas_call(
        _att_body,
        grid=(n // br,),
        in_specs=[
            pl.BlockSpec((br, d), lambda i: (i, 0)),
            pl.BlockSpec((br, 1), lambda i: (i, 0)),
            pl.BlockSpec((d, 1), lambda i: (0, 0)),
        ],
        out_specs=pl.BlockSpec((br, d), lambda i: (i, 0)),
        out_shape=jax.ShapeDtypeStruct((n, d), jnp.float32),
    )(x, z, a1)


def _spmm(ei, w, x, n):
    return jax.ops.segment_sum(w[:, None] * x[ei[1]], ei[0], num_segments=n)


def _par1_mv(ei, xs, n):
    return jax.ops.segment_sum(xs, ei[1], num_segments=n) - jax.ops.segment_sum(
        xs, ei[0], num_segments=n
    )


def _par1t_mv(ei, xt):
    return xt[ei[1]] - xt[ei[0]]


def _degree(ei, n):
    return (
        jax.ops.segment_sum(
            jnp.ones((ei.size,), jnp.float32), ei.reshape(-1), num_segments=n
        )
        + 1e-6
    )


def _scatter_mean(x, idx, n):
    s = jax.ops.segment_sum(x, idx, num_segments=n)
    c = jax.ops.segment_sum(jnp.ones((x.shape[0],), jnp.float32), idx, num_segments=n)
    return s / jnp.maximum(c, 1.0)[:, None]


def _bn_coeffs(y, g, be):
    m = jnp.mean(y, axis=0)
    v = jnp.var(y, axis=0)
    a = g / jnp.sqrt(v + 1e-5)
    c = be - m * a
    return a, c


def _bn_relu(y, g, be):
    a, c = _bn_coeffs(y, g, be)
    return _paffine_relu(y, a, c)


def kernel(
    x_t,
    x_s,
    edge_index_t,
    edge_weight_t,
    edge_index_s,
    edge_weight_s,
    edge_index,
    pos_t,
    pos_s,
    edge_index_t1,
    edge_weight_t1,
    edge_index_s1,
    edge_weight_s1,
    edge_index1,
    n_batch1,
    s_batch1,
    params,
):
    nN, nE = x_t.shape[0], x_s.shape[0]

    p = params["init_t"]
    xt = _bn_relu(_pmm(x_t, p["W0"], p["b"]), p["g"], p["be"])
    p = params["init_s"]
    xs = _bn_relu(_pmm(x_s, p["W0"], p["b"]), p["g"], p["be"])
    xt0, xs0 = xt, xs

    ei = edge_index
    d_inv = 1.0 / _degree(ei, nN)
    eit, wt, eis, ws = edge_index_t, edge_weight_t, edge_index_s, edge_weight_s

    zero1 = jnp.zeros((1,), jnp.float32)
    for i, f in enumerate(_FILTERS):
        zf = jnp.zeros((f,), jnp.float32)
        for j in range(_CHANNELS[i]):
            q = params["neint%d%d" % (i, j)]
            u = _pmm(xs0, q["Wts"], zf)
            s2tf = _par1_mv(ei, u, nN) * d_inv[:, None]
            v = _pmm(xt0, q["Wst"], zf)
            t2sf = _par1t_mv(ei, v)
            xt = _pmm_add(xt0, q["Wt"], s2tf, q["bt"], relu=True)
            xs = _pmm_add(xs0, q["Ws"], t2sf, q["bs"], relu=True)

            q = params["convt%d%d" % (i, j)]
            lt = _spmm(eit, wt, xt, nN)
            yt = _pmm2(xt, q["W0"] + q["W1"], lt, -q["W1"], q["b"])
            xt = _bn_relu(yt, q["g"], q["be"])

            q = params["convs%d%d" % (i, j)]
            ls = _spmm(eis, ws, xs, nE)
            ys = _pmm2(xs, q["W0"] + q["W1"], ls, -q["W1"], q["b"])
            xs = _bn_relu(ys, q["g"], q["be"])

            xt0 = jnp.concatenate([xt0, xt], -1)
            xs0 = jnp.concatenate([xs0, xs], -1)

        q = params["neatt%d" % i]
        zt = _par1_mv(ei, _pmm(xs0, q["ats"], zero1), nN) * d_inv[:, None]
        zs = _par1t_mv(ei, _pmm(xt0, q["ast"], zero1))
        xt0 = _patt_scale(xt0, zt, q["at"])
        xs0 = _patt_scale(xs0, zs, q["as"])

        if i == 0:
            xt0 = _scatter_mean(xt0, pos_t, _N1)
            xs0 = _scatter_mean(xs0, pos_s, _E1)
            eit, wt, eis, ws = (
                edge_index_t1,
                edge_weight_t1,
                edge_index_s1,
                edge_weight_s1,
            )
            ei = edge_index1
            nN, nE = _N1, _E1
            d_inv = 1.0 / _degree(ei, nN)

    x = jnp.concatenate(
        [_scatter_mean(xs, s_batch1, _NB), _scatter_mean(xt, n_batch1, _NB)], -1
    )
    return _pmm(x, params["out"]["W"], params["out"]["b"])


# speedup vs baseline: 1.2335x; 1.2335x over previous
import functools

import jax
import jax.numpy as jnp
from jax.experimental import pallas as pl

_FILTERS = [64, 128, 256, 512]
_CHANNELS = [2, 2, 2, 2]
_N1, _E1, _NB = 2000, 8000, 8


def _mm1_body(x_ref, w_ref, b_ref, o_ref, *, relu):
    y = jnp.dot(x_ref[...], w_ref[...], preferred_element_type=jnp.float32)
    y = y + b_ref[...]
    if relu:
        y = jnp.maximum(y, 0.0)
    o_ref[...] = y


def _mm2_body(x1_ref, w1_ref, x2_ref, w2_ref, b_ref, o_ref, *, relu):
    y = jnp.dot(x1_ref[...], w1_ref[...], preferred_element_type=jnp.float32)
    y = y + jnp.dot(x2_ref[...], w2_ref[...], preferred_element_type=jnp.float32)
    y = y + b_ref[...]
    if relu:
        y = jnp.maximum(y, 0.0)
    o_ref[...] = y


def _mm_add_body(x_ref, w_ref, add_ref, b_ref, o_ref, *, relu):
    y = jnp.dot(x_ref[...], w_ref[...], preferred_element_type=jnp.float32)
    y = y + add_ref[...] + b_ref[...]
    if relu:
        y = jnp.maximum(y, 0.0)
    o_ref[...] = y


def _affine_relu_body(y_ref, a_ref, c_ref, o_ref):
    o_ref[...] = jnp.maximum(y_ref[...] * a_ref[...] + c_ref[...], 0.0)


def _att2_body(x_ref, s_ref, a1_ref, a2_ref, o_ref):
    logit = jnp.dot(x_ref[...], a1_ref[...], preferred_element_type=jnp.float32)
    logit = logit + jnp.dot(s_ref[...], a2_ref[...], preferred_element_type=jnp.float32)
    o_ref[...] = x_ref[...] * (0.5 + 0.5 * jax.nn.sigmoid(logit))


def _attz_body(x_ref, z_ref, a1_ref, o_ref):
    logit = jnp.dot(x_ref[...], a1_ref[...], preferred_element_type=jnp.float32)
    logit = logit + z_ref[...]
    o_ref[...] = x_ref[...] * (0.5 + 0.5 * jax.nn.sigmoid(logit))


def _pool_body(pos_ref, x_ref, o_ref, *, nout):
    @pl.when(pl.program_id(0) == 0)
    def _():
        o_ref[...] = jnp.zeros_like(o_ref)

    pos = pos_ref[0, 0, :]
    oh = (
        pos[None, :]
        == jax.lax.broadcasted_iota(jnp.int32, (nout, pos.shape[0]), 0)
    ).astype(jnp.float32)
    o_ref[...] += jnp.dot(oh, x_ref[...], preferred_element_type=jnp.float32)


def _rows_block(n, k=0):
    br = 1000 if n % 1000 == 0 else n
    if k >= 4000 and n % 400 == 0:
        br = 400
    return br


def _pmm(x, w, b, relu=False):
    n, k = x.shape
    m = w.shape[1]
    br = _rows_block(n, k)
    return pl.pallas_call(
        functools.partial(_mm1_body, relu=relu),
        grid=(n // br,),
        in_specs=[
            pl.BlockSpec((br, k), lambda i: (i, 0)),
            pl.BlockSpec((k, m), lambda i: (0, 0)),
            pl.BlockSpec((1, m), lambda i: (0, 0)),
        ],
        out_specs=pl.BlockSpec((br, m), lambda i: (i, 0)),
        out_shape=jax.ShapeDtypeStruct((n, m), jnp.float32),
    )(x, w, b.reshape(1, -1))


def _pmm2(x1, w1, x2, w2, b, relu=False):
    n, k1 = x1.shape
    k2 = x2.shape[1]
    m = w1.shape[1]
    br = _rows_block(n)
    return pl.pallas_call(
        functools.partial(_mm2_body, relu=relu),
        grid=(n // br,),
        in_specs=[
            pl.BlockSpec((br, k1), lambda i: (i, 0)),
            pl.BlockSpec((k1, m), lambda i: (0, 0)),
            pl.BlockSpec((br, k2), lambda i: (i, 0)),
            pl.BlockSpec((k2, m), lambda i: (0, 0)),
            pl.BlockSpec((1, m), lambda i: (0, 0)),
        ],
        out_specs=pl.BlockSpec((br, m), lambda i: (i, 0)),
        out_shape=jax.ShapeDtypeStruct((n, m), jnp.float32),
    )(x1, w1, x2, w2, b.reshape(1, -1))


def _pmm_add(x, w, add, b, relu=False):
    n, k = x.shape
    m = w.shape[1]
    br = _rows_block(n)
    return pl.pallas_call(
        functools.partial(_mm_add_body, relu=relu),
        grid=(n // br,),
        in_specs=[
            pl.BlockSpec((br, k), lambda i: (i, 0)),
            pl.BlockSpec((k, m), lambda i: (0, 0)),
            pl.BlockSpec((br, m), lambda i: (i, 0)),
            pl.BlockSpec((1, m), lambda i: (0, 0)),
        ],
        out_specs=pl.BlockSpec((br, m), lambda i: (i, 0)),
        out_shape=jax.ShapeDtypeStruct((n, m), jnp.float32),
    )(x, w, add, b.reshape(1, -1))


def _paffine_relu(y, a, c):
    n, m = y.shape
    br = _rows_block(n)
    return pl.pallas_call(
        _affine_relu_body,
        grid=(n // br,),
        in_specs=[
            pl.BlockSpec((br, m), lambda i: (i, 0)),
            pl.BlockSpec((1, m), lambda i: (0, 0)),
            pl.BlockSpec((1, m), lambda i: (0, 0)),
        ],
        out_specs=pl.BlockSpec((br, m), lambda i: (i, 0)),
        out_shape=jax.ShapeDtypeStruct((n, m), jnp.float32),
    )(y, a.reshape(1, -1), c.reshape(1, -1))


def _patt2(x, s, a1, a2):
    n, d = x.shape
    br = _rows_block(n)
    return pl.pallas_call(
        _att2_body,
        grid=(n // br,),
        in_specs=[
            pl.BlockSpec((br, d), lambda i: (i, 0)),
            pl.BlockSpec((br, d), lambda i: (i, 0)),
            pl.BlockSpec((d, 1), lambda i: (0, 0)),
            pl.BlockSpec((d, 1), lambda i: (0, 0)),
        ],
        out_specs=pl.BlockSpec((br, d), lambda i: (i, 0)),
        out_shape=jax.ShapeDtypeStruct((n, d), jnp.float32),
    )(x, s, a1, a2)


def _pattz(x, z, a1):
    n, d = x.shape
    br = _rows_block(n)
    return pl.pallas_call(
        _attz_body,
        grid=(n // br,),
        in_specs=[
            pl.BlockSpec((br, d), lambda i: (i, 0)),
            pl.BlockSpec((br, 1), lambda i: (i, 0)),
            pl.BlockSpec((d, 1), lambda i: (0, 0)),
        ],
        out_specs=pl.BlockSpec((br, d), lambda i: (i, 0)),
        out_shape=jax.ShapeDtypeStruct((n, d), jnp.float32),
    )(x, z, a1)


def _ppool_sum(x, pos, nout):
    n, d = x.shape
    br = _rows_block(n)
    return pl.pallas_call(
        functools.partial(_pool_body, nout=nout),
        grid=(n // br,),
        in_specs=[
            pl.BlockSpec((1, 1, br), lambda i: (i, 0, 0)),
            pl.BlockSpec((br, d), lambda i: (i, 0)),
        ],
        out_specs=pl.BlockSpec((nout, d), lambda i: (0, 0)),
        out_shape=jax.ShapeDtypeStruct((nout, d), jnp.float32),
    )(pos.reshape(n // br, 1, br), x)


def _spmm(ei, w, x, n):
    return jax.ops.segment_sum(w[:, None] * x[ei[1]], ei[0], num_segments=n)


def _par1_mv(ei, xs, n):
    return jax.ops.segment_sum(xs, ei[1], num_segments=n) - jax.ops.segment_sum(
        xs, ei[0], num_segments=n
    )


def _par1t_mv(ei, xt):
    return xt[ei[1]] - xt[ei[0]]


def _degree(ei, n):
    return (
        jax.ops.segment_sum(
            jnp.ones((ei.size,), jnp.float32), ei.reshape(-1), num_segments=n
        )
        + 1e-6
    )


def _counts(idx, n):
    return jax.ops.segment_sum(
        jnp.ones((idx.shape[0],), jnp.float32), idx, num_segments=n
    )


def _scatter_mean(x, idx, n):
    s = jax.ops.segment_sum(x, idx, num_segments=n)
    return s / jnp.maximum(_counts(idx, n), 1.0)[:, None]


def _pool_mean(x, idx, n):
    s = _ppool_sum(x, idx, n)
    return s / jnp.maximum(_counts(idx, n), 1.0)[:, None]


def _bn_relu(y, g, be):
    m = jnp.mean(y, axis=0)
    v = jnp.var(y, axis=0)
    a = g / jnp.sqrt(v + 1e-5)
    c = be - m * a
    return _paffine_relu(y, a, c)


def kernel(
    x_t,
    x_s,
    edge_index_t,
    edge_weight_t,
    edge_index_s,
    edge_weight_s,
    edge_index,
    pos_t,
    pos_s,
    edge_index_t1,
    edge_weight_t1,
    edge_index_s1,
    edge_weight_s1,
    edge_index1,
    n_batch1,
    s_batch1,
    params,
):
    nN, nE = x_t.shape[0], x_s.shape[0]

    lt1 = (
        jnp.zeros((_N1, _N1), jnp.float32)
        .at[edge_index_t1[0], edge_index_t1[1]]
        .add(edge_weight_t1)
    )
    ls1 = (
        jnp.zeros((_E1, _E1), jnp.float32)
        .at[edge_index_s1[0], edge_index_s1[1]]
        .add(edge_weight_s1)
    )
    ar_e1 = jnp.arange(_E1)
    b1 = (
        jnp.zeros((_N1, _E1), jnp.float32)
        .at[edge_index1[1], ar_e1]
        .add(1.0)
        .at[edge_index1[0], ar_e1]
        .add(-1.0)
    )
    b1t = b1.T

    p = params["init_t"]
    xt = _bn_relu(_pmm(x_t, p["W0"], p["b"]), p["g"], p["be"])
    p = params["init_s"]
    xs = _bn_relu(_pmm(x_s, p["W0"], p["b"]), p["g"], p["be"])
    xt0, xs0 = xt, xs

    ei = edge_index
    d_inv = 1.0 / _degree(ei, nN)
    eit, wt, eis, ws = edge_index_t, edge_weight_t, edge_index_s, edge_weight_s

    zero1 = jnp.zeros((1,), jnp.float32)
    fine = True
    for i, f in enumerate(_FILTERS):
        zf = jnp.zeros((f,), jnp.float32)
        for j in range(_CHANNELS[i]):
            q = params["neint%d%d" % (i, j)]
            u = _pmm(xs0, q["Wts"], zf)
            v = _pmm(xt0, q["Wst"], zf)
            if fine:
                s2tf = _par1_mv(ei, u, nN) * d_inv[:, None]
                t2sf = _par1t_mv(ei, v)
            else:
                s2tf = _pmm(b1, u, zf) * d_inv[:, None]
                t2sf = _pmm(b1t, v, zf)
            xt = _pmm_add(xt0, q["Wt"], s2tf, q["bt"], relu=True)
            xs = _pmm_add(xs0, q["Ws"], t2sf, q["bs"], relu=True)

            q = params["convt%d%d" % (i, j)]
            lt = _spmm(eit, wt, xt, nN) if fine else _pmm(lt1, xt, zf)
            yt = _pmm2(xt, q["W0"] + q["W1"], lt, -q["W1"], q["b"])
            xt = _bn_relu(yt, q["g"], q["be"])

            q = params["convs%d%d" % (i, j)]
            ls = _spmm(eis, ws, xs, nE) if fine else _pmm(ls1, xs, zf)
            ys = _pmm2(xs, q["W0"] + q["W1"], ls, -q["W1"], q["b"])
            xs = _bn_relu(ys, q["g"], q["be"])

            xt0 = jnp.concatenate([xt0, xt], -1)
            xs0 = jnp.concatenate([xs0, xs], -1)

        q = params["neatt%d" % i]
        us = _pmm(xs0, q["ats"], zero1)
        vs = _pmm(xt0, q["ast"], zero1)
        if fine:
            zt = _par1_mv(ei, us, nN) * d_inv[:, None]
            zs = _par1t_mv(ei, vs)
        else:
            zt = _pmm(b1, us, zero1) * d_inv[:, None]
            zs = _pmm(b1t, vs, zero1)
        xt0 = _pattz(xt0, zt, q["at"])
        xs0 = _pattz(xs0, zs, q["as"])

        if i == 0:
            xt0 = _pool_mean(xt0, pos_t, _N1)
            xs0 = _scatter_mean(xs0, pos_s, _E1)
            ei = edge_index1
            nN, nE = _N1, _E1
            d_inv = 1.0 / _degree(ei, nN)
            fine = False

    x = jnp.concatenate(
        [_pool_mean(xs, s_batch1, _NB), _pool_mean(xt, n_batch1, _NB)], -1
    )
    return _pmm(x, params["out"]["W"], params["out"]["b"])


# speedup vs baseline: 1.4425x; 1.1694x over previous
import functools

import jax
import jax.numpy as jnp
from jax.experimental import pallas as pl

_FILTERS = [64, 128, 256, 512]
_CHANNELS = [2, 2, 2, 2]
_N1, _E1, _NB = 2000, 8000, 8


def _mm1_body(x_ref, w_ref, b_ref, o_ref, *, relu):
    y = jnp.dot(x_ref[...], w_ref[...], preferred_element_type=jnp.float32)
    y = y + b_ref[...]
    if relu:
        y = jnp.maximum(y, 0.0)
    o_ref[...] = y


def _mm2_body(x1_ref, w1_ref, x2_ref, w2_ref, b_ref, o_ref, *, relu):
    y = jnp.dot(x1_ref[...], w1_ref[...], preferred_element_type=jnp.float32)
    y = y + jnp.dot(x2_ref[...], w2_ref[...], preferred_element_type=jnp.float32)
    y = y + b_ref[...]
    if relu:
        y = jnp.maximum(y, 0.0)
    o_ref[...] = y


def _mm_add_body(x_ref, w_ref, add_ref, b_ref, o_ref, *, relu):
    y = jnp.dot(x_ref[...], w_ref[...], preferred_element_type=jnp.float32)
    y = y + add_ref[...] + b_ref[...]
    if relu:
        y = jnp.maximum(y, 0.0)
    o_ref[...] = y


def _affine_relu_body(y_ref, a_ref, c_ref, o_ref):
    o_ref[...] = jnp.maximum(y_ref[...] * a_ref[...] + c_ref[...], 0.0)


def _att2_body(x_ref, s_ref, a1_ref, a2_ref, o_ref):
    logit = jnp.dot(x_ref[...], a1_ref[...], preferred_element_type=jnp.float32)
    logit = logit + jnp.dot(s_ref[...], a2_ref[...], preferred_element_type=jnp.float32)
    o_ref[...] = x_ref[...] * (0.5 + 0.5 * jax.nn.sigmoid(logit))


def _attz_body(x_ref, z_ref, a1_ref, o_ref):
    logit = jnp.dot(x_ref[...], a1_ref[...], preferred_element_type=jnp.float32)
    logit = logit + z_ref[...]
    o_ref[...] = x_ref[...] * (0.5 + 0.5 * jax.nn.sigmoid(logit))


def _pool_body(pos_ref, x_ref, o_ref, *, nout):
    @pl.when(pl.program_id(0) == 0)
    def _():
        o_ref[...] = jnp.zeros_like(o_ref)

    pos = pos_ref[0, 0, :]
    oh = (
        pos[None, :]
        == jax.lax.broadcasted_iota(jnp.int32, (nout, pos.shape[0]), 0)
    ).astype(jnp.float32)
    o_ref[...] += jnp.dot(oh, x_ref[...], preferred_element_type=jnp.float32)


def _rows_block(n, k=0):
    br = 1000 if n % 1000 == 0 else n
    if k >= 4000 and n % 400 == 0:
        br = 400
    return br


def _pmm(x, w, b, relu=False):
    n, k = x.shape
    m = w.shape[1]
    br = _rows_block(n, k)
    return pl.pallas_call(
        functools.partial(_mm1_body, relu=relu),
        grid=(n // br,),
        in_specs=[
            pl.BlockSpec((br, k), lambda i: (i, 0)),
            pl.BlockSpec((k, m), lambda i: (0, 0)),
            pl.BlockSpec((1, m), lambda i: (0, 0)),
        ],
        out_specs=pl.BlockSpec((br, m), lambda i: (i, 0)),
        out_shape=jax.ShapeDtypeStruct((n, m), jnp.float32),
    )(x, w, b.reshape(1, -1))


def _pmm2(x1, w1, x2, w2, b, relu=False):
    n, k1 = x1.shape
    k2 = x2.shape[1]
    m = w1.shape[1]
    br = _rows_block(n)
    return pl.pallas_call(
        functools.partial(_mm2_body, relu=relu),
        grid=(n // br,),
        in_specs=[
            pl.BlockSpec((br, k1), lambda i: (i, 0)),
            pl.BlockSpec((k1, m), lambda i: (0, 0)),
            pl.BlockSpec((br, k2), lambda i: (i, 0)),
            pl.BlockSpec((k2, m), lambda i: (0, 0)),
            pl.BlockSpec((1, m), lambda i: (0, 0)),
        ],
        out_specs=pl.BlockSpec((br, m), lambda i: (i, 0)),
        out_shape=jax.ShapeDtypeStruct((n, m), jnp.float32),
    )(x1, w1, x2, w2, b.reshape(1, -1))


def _pmm_add(x, w, add, b, relu=False):
    n, k = x.shape
    m = w.shape[1]
    br = _rows_block(n)
    return pl.pallas_call(
        functools.partial(_mm_add_body, relu=relu),
        grid=(n // br,),
        in_specs=[
            pl.BlockSpec((br, k), lambda i: (i, 0)),
            pl.BlockSpec((k, m), lambda i: (0, 0)),
            pl.BlockSpec((br, m), lambda i: (i, 0)),
            pl.BlockSpec((1, m), lambda i: (0, 0)),
        ],
        out_specs=pl.BlockSpec((br, m), lambda i: (i, 0)),
        out_shape=jax.ShapeDtypeStruct((n, m), jnp.float32),
    )(x, w, add, b.reshape(1, -1))


def _paffine_relu(y, a, c):
    n, m = y.shape
    br = _rows_block(n)
    return pl.pallas_call(
        _affine_relu_body,
        grid=(n // br,),
        in_specs=[
            pl.BlockSpec((br, m), lambda i: (i, 0)),
            pl.BlockSpec((1, m), lambda i: (0, 0)),
            pl.BlockSpec((1, m), lambda i: (0, 0)),
        ],
        out_specs=pl.BlockSpec((br, m), lambda i: (i, 0)),
        out_shape=jax.ShapeDtypeStruct((n, m), jnp.float32),
    )(y, a.reshape(1, -1), c.reshape(1, -1))


def _patt2(x, s, a1, a2):
    n, d = x.shape
    br = _rows_block(n)
    return pl.pallas_call(
        _att2_body,
        grid=(n // br,),
        in_specs=[
            pl.BlockSpec((br, d), lambda i: (i, 0)),
            pl.BlockSpec((br, d), lambda i: (i, 0)),
            pl.BlockSpec((d, 1), lambda i: (0, 0)),
            pl.BlockSpec((d, 1), lambda i: (0, 0)),
        ],
        out_specs=pl.BlockSpec((br, d), lambda i: (i, 0)),
        out_shape=jax.ShapeDtypeStruct((n, d), jnp.float32),
    )(x, s, a1, a2)


def _pattz(x, z, a1):
    n, d = x.shape
    br = _rows_block(n)
    return pl.pallas_call(
        _attz_body,
        grid=(n // br,),
        in_specs=[
            pl.BlockSpec((br, d), lambda i: (i, 0)),
            pl.BlockSpec((br, 1), lambda i: (i, 0)),
            pl.BlockSpec((d, 1), lambda i: (0, 0)),
        ],
        out_specs=pl.BlockSpec((br, d), lambda i: (i, 0)),
        out_shape=jax.ShapeDtypeStruct((n, d), jnp.float32),
    )(x, z, a1)


def _ppool_sum(x, pos, nout):
    n, d = x.shape
    br = _rows_block(n)
    return pl.pallas_call(
        functools.partial(_pool_body, nout=nout),
        grid=(n // br,),
        in_specs=[
            pl.BlockSpec((1, 1, br), lambda i: (i, 0, 0)),
            pl.BlockSpec((br, d), lambda i: (i, 0)),
        ],
        out_specs=pl.BlockSpec((nout, d), lambda i: (0, 0)),
        out_shape=jax.ShapeDtypeStruct((nout, d), jnp.float32),
    )(pos.reshape(n // br, 1, br), x)


def _spmm(ei, w, x, n):
    return jax.ops.segment_sum(w[:, None] * x[ei[1]], ei[0], num_segments=n)


def _par1_mv(ei, xs, n):
    return jax.ops.segment_sum(xs, ei[1], num_segments=n) - jax.ops.segment_sum(
        xs, ei[0], num_segments=n
    )


def _par1t_mv(ei, xt):
    return xt[ei[1]] - xt[ei[0]]


def _degree(ei, n):
    return (
        jax.ops.segment_sum(
            jnp.ones((ei.size,), jnp.float32), ei.reshape(-1), num_segments=n
        )
        + 1e-6
    )


def _counts(idx, n):
    return jax.ops.segment_sum(
        jnp.ones((idx.shape[0],), jnp.float32), idx, num_segments=n
    )


def _scatter_mean(x, idx, n):
    s = jax.ops.segment_sum(x, idx, num_segments=n)
    return s / jnp.maximum(_counts(idx, n), 1.0)[:, None]


def _pool_mean(x, idx, n):
    s = _ppool_sum(x, idx, n)
    return s / jnp.maximum(_counts(idx, n), 1.0)[:, None]


def _bn_relu(y, g, be):
    m = jnp.mean(y, axis=0)
    v = jnp.var(y, axis=0)
    a = g / jnp.sqrt(v + 1e-5)
    c = be - m * a
    return _paffine_relu(y, a, c)


def kernel(
    x_t,
    x_s,
    edge_index_t,
    edge_weight_t,
    edge_index_s,
    edge_weight_s,
    edge_index,
    pos_t,
    pos_s,
    edge_index_t1,
    edge_weight_t1,
    edge_index_s1,
    edge_weight_s1,
    edge_index1,
    n_batch1,
    s_batch1,
    params,
):
    nN, nE = x_t.shape[0], x_s.shape[0]

    lt1 = (
        jnp.zeros((_N1, _N1), jnp.float32)
        .at[edge_index_t1[0], edge_index_t1[1]]
        .add(edge_weight_t1)
    )
    ls1 = (
        jnp.zeros((_E1, _E1), jnp.float32)
        .at[edge_index_s1[0], edge_index_s1[1]]
        .add(edge_weight_s1)
    )
    ar_e1 = jnp.arange(_E1)
    b1 = (
        jnp.zeros((_N1, _E1), jnp.float32)
        .at[edge_index1[1], ar_e1]
        .add(1.0)
        .at[edge_index1[0], ar_e1]
        .add(-1.0)
    )
    b1t = b1.T

    p = params["init_t"]
    xt = _bn_relu(_pmm(x_t, p["W0"], p["b"]), p["g"], p["be"])
    p = params["init_s"]
    xs = _bn_relu(_pmm(x_s, p["W0"], p["b"]), p["g"], p["be"])
    xt0, xs0 = xt, xs

    ei = edge_index
    d_inv = 1.0 / _degree(ei, nN)
    eit, wt, eis, ws = edge_index_t, edge_weight_t, edge_index_s, edge_weight_s

    zero1 = jnp.zeros((1,), jnp.float32)
    fine = True
    for i, f in enumerate(_FILTERS):
        zf = jnp.zeros((f,), jnp.float32)
        for j in range(_CHANNELS[i]):
            q = params["neint%d%d" % (i, j)]
            if fine:
                s2t = _par1_mv(ei, xs0, nN) * d_inv[:, None]
                t2s = _par1t_mv(ei, xt0)
                xt = _pmm2(xt0, q["Wt"], s2t, q["Wts"], q["bt"], relu=True)
                xs = _pmm2(xs0, q["Ws"], t2s, q["Wst"], q["bs"], relu=True)
            else:
                u = _pmm(xs0, q["Wts"], zf)
                v = _pmm(xt0, q["Wst"], zf)
                s2tf = _pmm(b1, u, zf) * d_inv[:, None]
                t2sf = _pmm(b1t, v, zf)
                xt = _pmm_add(xt0, q["Wt"], s2tf, q["bt"], relu=True)
                xs = _pmm_add(xs0, q["Ws"], t2sf, q["bs"], relu=True)

            q = params["convt%d%d" % (i, j)]
            lt = _spmm(eit, wt, xt, nN) if fine else _pmm(lt1, xt, zf)
            yt = _pmm2(xt, q["W0"] + q["W1"], lt, -q["W1"], q["b"])
            xt = _bn_relu(yt, q["g"], q["be"])

            q = params["convs%d%d" % (i, j)]
            ls = _spmm(eis, ws, xs, nE) if fine else _pmm(ls1, xs, zf)
            ys = _pmm2(xs, q["W0"] + q["W1"], ls, -q["W1"], q["b"])
            xs = _bn_relu(ys, q["g"], q["be"])

            xt0 = jnp.concatenate([xt0, xt], -1)
            xs0 = jnp.concatenate([xs0, xs], -1)

        q = params["neatt%d" % i]
        if fine:
            s2t = _par1_mv(ei, xs0, nN) * d_inv[:, None]
            t2s = _par1t_mv(ei, xt0)
            xt0 = _patt2(xt0, s2t, q["at"], q["ats"])
            xs0 = _patt2(xs0, t2s, q["as"], q["ast"])
        else:
            us = _pmm(xs0, q["ats"], zero1)
            vs = _pmm(xt0, q["ast"], zero1)
            zt = _pmm(b1, us, zero1) * d_inv[:, None]
            zs = _pmm(b1t, vs, zero1)
            xt0 = _pattz(xt0, zt, q["at"])
            xs0 = _pattz(xs0, zs, q["as"])

        if i == 0:
            xt0 = _pool_mean(xt0, pos_t, _N1)
            xs0 = _scatter_mean(xs0, pos_s, _E1)
            ei = edge_index1
            nN, nE = _N1, _E1
            d_inv = 1.0 / _degree(ei, nN)
            fine = False

    x = jnp.concatenate(
        [_pool_mean(xs, s_batch1, _NB), _pool_mean(xt, n_batch1, _NB)], -1
    )
    return _pmm(x, params["out"]["W"], params["out"]["b"])
